# baseline (speedup 1.0000x reference)
import functools

import jax
import jax.numpy as jnp
from jax import lax
from jax.experimental import pallas as pl
from jax.experimental.pallas import tpu as pltpu
from jax.experimental.pallas import tpu_sc as plsc

_VOCAB = 100000
_HID = 64
_BATCH = 4
_SEQ = 512
_NROWS = _BATCH * _SEQ

_NC = 2
_NS = 16
_NW = _NC * _NS
_RPW = _NROWS // _NW

_BM = 4096


def _gather_pe_sc(ids_flat, pe, emb):
    mesh = plsc.VectorSubcoreMesh(core_axis_name="c", subcore_axis_name="s")

    @functools.partial(
        pl.kernel,
        mesh=mesh,
        out_type=jax.ShapeDtypeStruct((_NROWS, _HID), jnp.float32),
        scratch_types=[
            pltpu.VMEM((_RPW,), jnp.int32),
            pltpu.VMEM((_RPW, _HID), jnp.float32),
            pltpu.SemaphoreType.DMA,
        ],
        compiler_params=pltpu.CompilerParams(use_tc_tiling_on_sc=False),
    )
    def sc_kernel(ids_hbm, pe_hbm, emb_hbm, out_hbm, idx_v, rows_v, sem):
        wid = lax.axis_index("s") * _NC + lax.axis_index("c")
        base = wid * _RPW
        pltpu.sync_copy(ids_hbm.at[pl.ds(base, _RPW)], idx_v)
        pltpu.sync_copy(pe_hbm.at[pl.ds(lax.rem(base, _SEQ), _RPW)], rows_v)
        pltpu.async_copy(emb_hbm.at[idx_v], rows_v, sem, add=True).wait()
        pltpu.sync_copy(rows_v, out_hbm.at[pl.ds(base, _RPW)])

    return sc_kernel(ids_flat, pe, emb)


def _project_body(x_ref, wt_ref, o_ref):
    b = pl.program_id(1)
    xb = x_ref[pl.ds(b * _SEQ, _SEQ), :]
    o_ref[...] = lax.dot_general(
        wt_ref[...],
        xb,
        dimension_numbers=(((0,), (1,)), ((), ())),
        preferred_element_type=jnp.float32,
    )[None]


def _project_tc(xpe, WT):
    return pl.pallas_call(
        _project_body,
        grid=(pl.cdiv(_VOCAB, _BM), _BATCH),
        in_specs=[
            pl.BlockSpec((_NROWS, _HID), lambda j, b: (0, 0)),
            pl.BlockSpec((_HID, _BM), lambda j, b: (0, j)),
        ],
        out_specs=pl.BlockSpec((1, _BM, _SEQ), lambda j, b: (b, j, 0)),
        out_shape=jax.ShapeDtypeStruct((_BATCH, _VOCAB, _SEQ), jnp.float32),
        compiler_params=pltpu.CompilerParams(
            dimension_semantics=("arbitrary", "arbitrary"),
        ),
    )(xpe, WT)


def kernel(ids, emb, W, pe):
    ids_flat = ids.reshape(_NROWS)
    xpe = _gather_pe_sc(ids_flat, pe, emb)
    out_t = _project_tc(xpe, W.T)
    return jnp.transpose(out_t, (0, 2, 1))


# speedup vs baseline: 1.0436x; 1.0436x over previous
import functools

import jax
import jax.numpy as jnp
from jax import lax
from jax.experimental import pallas as pl
from jax.experimental.pallas import tpu as pltpu
from jax.experimental.pallas import tpu_sc as plsc

_VOCAB = 100000
_HID = 64
_LANES = 128
_BATCH = 4
_SEQ = 512
_NROWS = _BATCH * _SEQ

_NC = 2
_NS = 16
_NW = _NC * _NS
_RPW = _NROWS // _NW

_BT = 2048
_BM = 4096


def _transpose_pad_body(et_ref, o_ref):
    o_ref[:, : _HID] = et_ref[...].T
    o_ref[:, _HID:] = jnp.zeros((_BT, _LANES - _HID), jnp.float32)


def _transpose_pad_tc(embT):
    return pl.pallas_call(
        _transpose_pad_body,
        grid=(pl.cdiv(_VOCAB, _BT),),
        in_specs=[pl.BlockSpec((_HID, _BT), lambda j: (0, j))],
        out_specs=pl.BlockSpec((_BT, _LANES), lambda j: (j, 0)),
        out_shape=jax.ShapeDtypeStruct((_VOCAB, _LANES), jnp.float32),
        compiler_params=pltpu.CompilerParams(
            dimension_semantics=("arbitrary",),
        ),
    )(embT)


def _gather_pe_sc(ids_flat, pe128, emb128):
    mesh = plsc.VectorSubcoreMesh(core_axis_name="c", subcore_axis_name="s")

    @functools.partial(
        pl.kernel,
        mesh=mesh,
        out_type=jax.ShapeDtypeStruct((_NROWS, _LANES), jnp.float32),
        scratch_types=[
            pltpu.VMEM((_RPW,), jnp.int32),
            pltpu.VMEM((_RPW, _LANES), jnp.float32),
            pltpu.SemaphoreType.DMA,
        ],
        compiler_params=pltpu.CompilerParams(use_tc_tiling_on_sc=True),
    )
    def sc_kernel(ids_hbm, pe_hbm, emb_hbm, out_hbm, idx_v, rows_v, sem):
        wid = lax.axis_index("s") * _NC + lax.axis_index("c")
        base = wid * _RPW
        pltpu.sync_copy(ids_hbm.at[pl.ds(base, _RPW)], idx_v)
        pltpu.sync_copy(pe_hbm.at[pl.ds(base, _RPW)], rows_v)
        pltpu.async_copy(emb_hbm.at[idx_v], rows_v, sem, add=True).wait()
        pltpu.sync_copy(rows_v, out_hbm.at[pl.ds(base, _RPW)])

    return sc_kernel(ids_flat, pe128, emb128)


def _project_body(x_ref, wt_ref, o_ref):
    b = pl.program_id(1)
    xb = x_ref[pl.ds(b * _SEQ, _SEQ), :_HID]
    o_ref[...] = lax.dot_general(
        wt_ref[...],
        xb,
        dimension_numbers=(((0,), (1,)), ((), ())),
        preferred_element_type=jnp.float32,
    )[None]


def _project_tc(xpe, WT):
    return pl.pallas_call(
        _project_body,
        grid=(pl.cdiv(_VOCAB, _BM), _BATCH),
        in_specs=[
            pl.BlockSpec((_NROWS, _LANES), lambda j, b: (0, 0)),
            pl.BlockSpec((_HID, _BM), lambda j, b: (0, j)),
        ],
        out_specs=pl.BlockSpec((1, _BM, _SEQ), lambda j, b: (b, j, 0)),
        out_shape=jax.ShapeDtypeStruct((_BATCH, _VOCAB, _SEQ), jnp.float32),
        compiler_params=pltpu.CompilerParams(
            dimension_semantics=("arbitrary", "arbitrary"),
        ),
    )(xpe, WT)


def kernel(ids, emb, W, pe):
    ids_flat = ids.reshape(_NROWS)
    emb128 = _transpose_pad_tc(emb.T)
    pe128 = jnp.pad(jnp.tile(pe, (_BATCH, 1)), ((0, 0), (0, _LANES - _HID)))
    xpe = _gather_pe_sc(ids_flat, pe128, emb128)
    out_t = _project_tc(xpe, W.T)
    return jnp.transpose(out_t, (0, 2, 1))


# speedup vs baseline: 1.0500x; 1.0061x over previous
import functools

import jax
import jax.numpy as jnp
from jax import lax
from jax.experimental import pallas as pl
from jax.experimental.pallas import tpu as pltpu
from jax.experimental.pallas import tpu_sc as plsc

_VOCAB = 100000
_HID = 64
_LANES = 128
_BATCH = 4
_SEQ = 512
_NROWS = _BATCH * _SEQ

_NC = 2
_NS = 16
_NW = _NC * _NS
_RPW = _NROWS // _NW

_BT = 2048
_BM = 8192


def _transpose_pad_body(et_ref, o_ref):
    o_ref[:, : _HID] = et_ref[...].T
    o_ref[:, _HID:] = jnp.zeros((_BT, _LANES - _HID), jnp.float32)


def _transpose_pad_tc(embT):
    return pl.pallas_call(
        _transpose_pad_body,
        grid=(pl.cdiv(_VOCAB, _BT),),
        in_specs=[pl.BlockSpec((_HID, _BT), lambda j: (0, j))],
        out_specs=pl.BlockSpec((_BT, _LANES), lambda j: (j, 0)),
        out_shape=jax.ShapeDtypeStruct((_VOCAB, _LANES), jnp.float32),
        compiler_params=pltpu.CompilerParams(
            dimension_semantics=("arbitrary",),
        ),
    )(embT)


def _gather_pe_sc(ids_flat, pe128, emb128):
    mesh = plsc.VectorSubcoreMesh(core_axis_name="c", subcore_axis_name="s")

    @functools.partial(
        pl.kernel,
        mesh=mesh,
        out_type=jax.ShapeDtypeStruct((_NROWS, _LANES), jnp.float32),
        scratch_types=[
            pltpu.VMEM((_RPW,), jnp.int32),
            pltpu.VMEM((_RPW, _LANES), jnp.float32),
            pltpu.SemaphoreType.DMA,
        ],
        compiler_params=pltpu.CompilerParams(use_tc_tiling_on_sc=True),
    )
    def sc_kernel(ids_hbm, pe_hbm, emb_hbm, out_hbm, idx_v, rows_v, sem):
        wid = lax.axis_index("s") * _NC + lax.axis_index("c")
        base = wid * _RPW
        pltpu.sync_copy(ids_hbm.at[pl.ds(base, _RPW)], idx_v)
        pltpu.sync_copy(pe_hbm.at[pl.ds(base, _RPW)], rows_v)
        pltpu.async_copy(emb_hbm.at[idx_v], rows_v, sem, add=True).wait()
        pltpu.sync_copy(rows_v, out_hbm.at[pl.ds(base, _RPW)])

    return sc_kernel(ids_flat, pe128, emb128)


def _project_body(x_ref, wt_ref, o_ref):
    b = pl.program_id(1)
    xb = x_ref[pl.ds(b * _SEQ, _SEQ), :_HID]
    o_ref[...] = lax.dot_general(
        wt_ref[...],
        xb,
        dimension_numbers=(((0,), (1,)), ((), ())),
        preferred_element_type=jnp.float32,
    )[None]


def _project_tc(xpe, WT):
    return pl.pallas_call(
        _project_body,
        grid=(pl.cdiv(_VOCAB, _BM), _BATCH),
        in_specs=[
            pl.BlockSpec((_NROWS, _LANES), lambda j, b: (0, 0)),
            pl.BlockSpec((_HID, _BM), lambda j, b: (0, j)),
        ],
        out_specs=pl.BlockSpec((1, _BM, _SEQ), lambda j, b: (b, j, 0)),
        out_shape=jax.ShapeDtypeStruct((_BATCH, _VOCAB, _SEQ), jnp.float32),
        compiler_params=pltpu.CompilerParams(
            dimension_semantics=("arbitrary", "arbitrary"),
        ),
    )(xpe, WT)


def kernel(ids, emb, W, pe):
    ids_flat = ids.reshape(_NROWS)
    emb128 = _transpose_pad_tc(emb.T)
    pe128 = jnp.pad(jnp.tile(pe, (_BATCH, 1)), ((0, 0), (0, _LANES - _HID)))
    xpe = _gather_pe_sc(ids_flat, pe128, emb128)
    out_t = _project_tc(xpe, W.T)
    return jnp.transpose(out_t, (0, 2, 1))


# speedup vs baseline: 1.0506x; 1.0007x over previous
import functools

import jax
import jax.numpy as jnp
from jax import lax
from jax.experimental import pallas as pl
from jax.experimental.pallas import tpu as pltpu
from jax.experimental.pallas import tpu_sc as plsc

_VOCAB = 100000
_HID = 64
_LANES = 128
_BATCH = 4
_SEQ = 512
_NROWS = _BATCH * _SEQ

_NC = 2
_NS = 16
_NW = _NC * _NS
_RPW = _NROWS // _NW

_BT = 2048
_BM = 8192


def _transpose_pad_body(et_ref, o_ref):
    o_ref[:, : _HID] = et_ref[...].T
    o_ref[:, _HID:] = jnp.zeros((_BT, _LANES - _HID), jnp.float32)


def _transpose_pad_tc(embT):
    return pl.pallas_call(
        _transpose_pad_body,
        grid=(pl.cdiv(_VOCAB, _BT),),
        in_specs=[pl.BlockSpec((_HID, _BT), lambda j: (0, j))],
        out_specs=pl.BlockSpec((_BT, _LANES), lambda j: (j, 0)),
        out_shape=jax.ShapeDtypeStruct((_VOCAB, _LANES), jnp.float32),
        compiler_params=pltpu.CompilerParams(
            dimension_semantics=("arbitrary",),
        ),
    )(embT)


def _gather_pe_sc(ids_flat, pe128, emb128):
    mesh = plsc.VectorSubcoreMesh(core_axis_name="c", subcore_axis_name="s")

    @functools.partial(
        pl.kernel,
        mesh=mesh,
        out_type=jax.ShapeDtypeStruct((_NROWS, _LANES), jnp.float32),
        scratch_types=[
            pltpu.VMEM((_RPW,), jnp.int32),
            pltpu.VMEM((_RPW, _LANES), jnp.float32),
            pltpu.SemaphoreType.DMA,
        ],
        compiler_params=pltpu.CompilerParams(use_tc_tiling_on_sc=True),
    )
    def sc_kernel(ids_hbm, pe_hbm, emb_hbm, out_hbm, idx_v, rows_v, sem):
        wid = lax.axis_index("s") * _NC + lax.axis_index("c")
        base = wid * _RPW
        pltpu.sync_copy(ids_hbm.at[pl.ds(base, _RPW)], idx_v)
        pltpu.sync_copy(pe_hbm.at[pl.ds(base, _RPW)], rows_v)
        pltpu.async_copy(emb_hbm.at[idx_v], rows_v, sem, add=True).wait()
        pltpu.sync_copy(rows_v, out_hbm.at[pl.ds(base, _RPW)])

    return sc_kernel(ids_flat, pe128, emb128)


def _project_body(x_ref, wt_ref, o_ref):
    b = pl.program_id(1)
    xb = x_ref[pl.ds(b * _SEQ, _SEQ), :_HID]
    o_ref[...] = lax.dot_general(
        wt_ref[...],
        xb,
        dimension_numbers=(((0,), (1,)), ((), ())),
        preferred_element_type=jnp.float32,
    )[None]


def _project_tc(xpe, WT):
    return pl.pallas_call(
        _project_body,
        grid=(pl.cdiv(_VOCAB, _BM), _BATCH),
        in_specs=[
            pl.BlockSpec((_NROWS, _LANES), lambda j, b: (0, 0)),
            pl.BlockSpec((_HID, _BM), lambda j, b: (0, j)),
        ],
        out_specs=pl.BlockSpec((1, _BM, _SEQ), lambda j, b: (b, j, 0)),
        out_shape=jax.ShapeDtypeStruct((_BATCH, _VOCAB, _SEQ), jnp.float32),
        compiler_params=pltpu.CompilerParams(
            dimension_semantics=("parallel", "parallel"),
        ),
    )(xpe, WT)


def kernel(ids, emb, W, pe):
    ids_flat = ids.reshape(_NROWS)
    emb128 = _transpose_pad_tc(emb.T)
    pe128 = jnp.pad(jnp.tile(pe, (_BATCH, 1)), ((0, 0), (0, _LANES - _HID)))
    xpe = _gather_pe_sc(ids_flat, pe128, emb128)
    out_t = _project_tc(xpe, W.T)
    return jnp.transpose(out_t, (0, 2, 1))


# speedup vs baseline: 1.1150x; 1.0613x over previous
import functools

import jax
import jax.numpy as jnp
from jax import lax
from jax.experimental import pallas as pl
from jax.experimental.pallas import tpu as pltpu
from jax.experimental.pallas import tpu_sc as plsc

_VOCAB = 100000
_HID = 64
_LANES = 128
_BATCH = 4
_SEQ = 512
_NROWS = _BATCH * _SEQ

_NC = 2
_NS = 16
_NW = _NC * _NS
_RPW = _NROWS // _NW

_BT = 8192
_BM = 8192


def _transpose_pad_body(et_ref, o_ref):
    o_ref[:, : _HID] = et_ref[...].T
    o_ref[:, _HID:] = jnp.zeros((_BT, _LANES - _HID), jnp.float32)


def _transpose_pad_tc(embT):
    return pl.pallas_call(
        _transpose_pad_body,
        grid=(pl.cdiv(_VOCAB, _BT),),
        in_specs=[pl.BlockSpec((_HID, _BT), lambda j: (0, j))],
        out_specs=pl.BlockSpec((_BT, _LANES), lambda j: (j, 0)),
        out_shape=jax.ShapeDtypeStruct((_VOCAB, _LANES), jnp.float32),
        compiler_params=pltpu.CompilerParams(
            dimension_semantics=("arbitrary",),
        ),
    )(embT)


def _gather_pe_sc(ids_flat, pe128, emb128):
    mesh = plsc.VectorSubcoreMesh(core_axis_name="c", subcore_axis_name="s")

    @functools.partial(
        pl.kernel,
        mesh=mesh,
        out_type=jax.ShapeDtypeStruct((_NROWS, _LANES), jnp.float32),
        scratch_types=[
            pltpu.VMEM((_RPW,), jnp.int32),
            pltpu.VMEM((_RPW, _LANES), jnp.float32),
            pltpu.SemaphoreType.DMA,
        ],
        compiler_params=pltpu.CompilerParams(use_tc_tiling_on_sc=True),
    )
    def sc_kernel(ids_hbm, pe_hbm, emb_hbm, out_hbm, idx_v, rows_v, sem):
        wid = lax.axis_index("s") * _NC + lax.axis_index("c")
        base = wid * _RPW
        pltpu.sync_copy(ids_hbm.at[pl.ds(base, _RPW)], idx_v)
        pltpu.sync_copy(pe_hbm.at[pl.ds(base, _RPW)], rows_v)
        pltpu.async_copy(emb_hbm.at[idx_v], rows_v, sem, add=True).wait()
        pltpu.sync_copy(rows_v, out_hbm.at[pl.ds(base, _RPW)])

    return sc_kernel(ids_flat, pe128, emb128)


def _project_body(x_ref, wt_ref, o_ref):
    b = pl.program_id(1)
    xb = x_ref[pl.ds(b * _SEQ, _SEQ), :_HID]
    o_ref[...] = lax.dot_general(
        wt_ref[...],
        xb,
        dimension_numbers=(((0,), (1,)), ((), ())),
        preferred_element_type=jnp.float32,
    )[None]


def _project_tc(xpe, WT):
    return pl.pallas_call(
        _project_body,
        grid=(pl.cdiv(_VOCAB, _BM), _BATCH),
        in_specs=[
            pl.BlockSpec((_NROWS, _LANES), lambda j, b: (0, 0)),
            pl.BlockSpec((_HID, _BM), lambda j, b: (0, j)),
        ],
        out_specs=pl.BlockSpec((1, _BM, _SEQ), lambda j, b: (b, j, 0)),
        out_shape=jax.ShapeDtypeStruct((_BATCH, _VOCAB, _SEQ), jnp.float32),
        compiler_params=pltpu.CompilerParams(
            dimension_semantics=("parallel", "parallel"),
        ),
    )(xpe, WT)


def kernel(ids, emb, W, pe):
    ids_flat = ids.reshape(_NROWS)
    emb128 = _transpose_pad_tc(emb.T)
    pe128 = jnp.pad(jnp.tile(pe, (_BATCH, 1)), ((0, 0), (0, _LANES - _HID)))
    xpe = _gather_pe_sc(ids_flat, pe128, emb128)
    out_t = _project_tc(xpe, W.T)
    return jnp.transpose(out_t, (0, 2, 1))


# speedup vs baseline: 1.1227x; 1.0069x over previous
import functools

import jax
import jax.numpy as jnp
from jax import lax
from jax.experimental import pallas as pl
from jax.experimental.pallas import tpu as pltpu
from jax.experimental.pallas import tpu_sc as plsc

_VOCAB = 100000
_HID = 64
_LANES = 128
_BATCH = 4
_SEQ = 512
_NROWS = _BATCH * _SEQ

_NC = 2
_NS = 16
_NW = _NC * _NS
_RPW = _NROWS // _NW

_BT = 16384
_BM = 8192


def _transpose_pad_body(et_ref, o_ref):
    o_ref[:, : _HID] = et_ref[...].T
    o_ref[:, _HID:] = jnp.zeros((_BT, _LANES - _HID), jnp.float32)


def _transpose_pad_tc(embT):
    return pl.pallas_call(
        _transpose_pad_body,
        grid=(pl.cdiv(_VOCAB, _BT),),
        in_specs=[pl.BlockSpec((_HID, _BT), lambda j: (0, j))],
        out_specs=pl.BlockSpec((_BT, _LANES), lambda j: (j, 0)),
        out_shape=jax.ShapeDtypeStruct((_VOCAB, _LANES), jnp.float32),
        compiler_params=pltpu.CompilerParams(
            dimension_semantics=("arbitrary",),
        ),
    )(embT)


def _gather_pe_sc(ids_flat, pe128, emb128):
    mesh = plsc.VectorSubcoreMesh(core_axis_name="c", subcore_axis_name="s")

    @functools.partial(
        pl.kernel,
        mesh=mesh,
        out_type=jax.ShapeDtypeStruct((_NROWS, _LANES), jnp.float32),
        scratch_types=[
            pltpu.VMEM((_RPW,), jnp.int32),
            pltpu.VMEM((_RPW, _LANES), jnp.float32),
            pltpu.SemaphoreType.DMA,
        ],
        compiler_params=pltpu.CompilerParams(use_tc_tiling_on_sc=True),
    )
    def sc_kernel(ids_hbm, pe_hbm, emb_hbm, out_hbm, idx_v, rows_v, sem):
        wid = lax.axis_index("s") * _NC + lax.axis_index("c")
        base = wid * _RPW
        pltpu.sync_copy(ids_hbm.at[pl.ds(base, _RPW)], idx_v)
        pltpu.sync_copy(pe_hbm.at[pl.ds(base, _RPW)], rows_v)
        pltpu.async_copy(emb_hbm.at[idx_v], rows_v, sem, add=True).wait()
        pltpu.sync_copy(rows_v, out_hbm.at[pl.ds(base, _RPW)])

    return sc_kernel(ids_flat, pe128, emb128)


def _project_body(x_ref, wt_ref, o_ref):
    b = pl.program_id(1)
    xb = x_ref[pl.ds(b * _SEQ, _SEQ), :_HID]
    o_ref[...] = lax.dot_general(
        wt_ref[...],
        xb,
        dimension_numbers=(((0,), (1,)), ((), ())),
        preferred_element_type=jnp.float32,
    )[None]


def _project_tc(xpe, WT):
    return pl.pallas_call(
        _project_body,
        grid=(pl.cdiv(_VOCAB, _BM), _BATCH),
        in_specs=[
            pl.BlockSpec((_NROWS, _LANES), lambda j, b: (0, 0)),
            pl.BlockSpec((_HID, _BM), lambda j, b: (0, j)),
        ],
        out_specs=pl.BlockSpec((1, _BM, _SEQ), lambda j, b: (b, j, 0)),
        out_shape=jax.ShapeDtypeStruct((_BATCH, _VOCAB, _SEQ), jnp.float32),
        compiler_params=pltpu.CompilerParams(
            dimension_semantics=("parallel", "parallel"),
        ),
    )(xpe, WT)


def kernel(ids, emb, W, pe):
    ids_flat = ids.reshape(_NROWS)
    emb128 = _transpose_pad_tc(emb.T)
    pe128 = jnp.pad(jnp.tile(pe, (_BATCH, 1)), ((0, 0), (0, _LANES - _HID)))
    xpe = _gather_pe_sc(ids_flat, pe128, emb128)
    out_t = _project_tc(xpe, W.T)
    return jnp.transpose(out_t, (0, 2, 1))


# speedup vs baseline: 1.1244x; 1.0015x over previous
import functools

import jax
import jax.numpy as jnp
from jax import lax
from jax.experimental import pallas as pl
from jax.experimental.pallas import tpu as pltpu
from jax.experimental.pallas import tpu_sc as plsc

_VOCAB = 100000
_HID = 64
_LANES = 128
_BATCH = 4
_SEQ = 512
_NROWS = _BATCH * _SEQ

_NC = 2
_NS = 16
_NW = _NC * _NS
_RPW = _NROWS // _NW

_BT = 32768
_BM = 8192


def _transpose_pad_body(et_ref, o_ref):
    o_ref[:, : _HID] = et_ref[...].T
    o_ref[:, _HID:] = jnp.zeros((_BT, _LANES - _HID), jnp.float32)


def _transpose_pad_tc(embT):
    return pl.pallas_call(
        _transpose_pad_body,
        grid=(pl.cdiv(_VOCAB, _BT),),
        in_specs=[pl.BlockSpec((_HID, _BT), lambda j: (0, j))],
        out_specs=pl.BlockSpec((_BT, _LANES), lambda j: (j, 0)),
        out_shape=jax.ShapeDtypeStruct((_VOCAB, _LANES), jnp.float32),
        compiler_params=pltpu.CompilerParams(
            dimension_semantics=("arbitrary",),
        ),
    )(embT)


def _gather_pe_sc(ids_flat, pe128, emb128):
    mesh = plsc.VectorSubcoreMesh(core_axis_name="c", subcore_axis_name="s")

    @functools.partial(
        pl.kernel,
        mesh=mesh,
        out_type=jax.ShapeDtypeStruct((_NROWS, _LANES), jnp.float32),
        scratch_types=[
            pltpu.VMEM((_RPW,), jnp.int32),
            pltpu.VMEM((_RPW, _LANES), jnp.float32),
            pltpu.SemaphoreType.DMA,
        ],
        compiler_params=pltpu.CompilerParams(use_tc_tiling_on_sc=True),
    )
    def sc_kernel(ids_hbm, pe_hbm, emb_hbm, out_hbm, idx_v, rows_v, sem):
        wid = lax.axis_index("s") * _NC + lax.axis_index("c")
        base = wid * _RPW
        pltpu.sync_copy(ids_hbm.at[pl.ds(base, _RPW)], idx_v)
        pltpu.sync_copy(pe_hbm.at[pl.ds(base, _RPW)], rows_v)
        pltpu.async_copy(emb_hbm.at[idx_v], rows_v, sem, add=True).wait()
        pltpu.sync_copy(rows_v, out_hbm.at[pl.ds(base, _RPW)])

    return sc_kernel(ids_flat, pe128, emb128)


def _project_body(x_ref, wt_ref, o_ref):
    b = pl.program_id(1)
    xb = x_ref[pl.ds(b * _SEQ, _SEQ), :_HID]
    o_ref[...] = lax.dot_general(
        wt_ref[...],
        xb,
        dimension_numbers=(((0,), (1,)), ((), ())),
        preferred_element_type=jnp.float32,
    )[None]


def _project_tc(xpe, WT):
    return pl.pallas_call(
        _project_body,
        grid=(pl.cdiv(_VOCAB, _BM), _BATCH),
        in_specs=[
            pl.BlockSpec((_NROWS, _LANES), lambda j, b: (0, 0)),
            pl.BlockSpec((_HID, _BM), lambda j, b: (0, j)),
        ],
        out_specs=pl.BlockSpec((1, _BM, _SEQ), lambda j, b: (b, j, 0)),
        out_shape=jax.ShapeDtypeStruct((_BATCH, _VOCAB, _SEQ), jnp.float32),
        compiler_params=pltpu.CompilerParams(
            dimension_semantics=("parallel", "parallel"),
        ),
    )(xpe, WT)


def kernel(ids, emb, W, pe):
    ids_flat = ids.reshape(_NROWS)
    emb128 = _transpose_pad_tc(emb.T)
    pe128 = jnp.pad(jnp.tile(pe, (_BATCH, 1)), ((0, 0), (0, _LANES - _HID)))
    xpe = _gather_pe_sc(ids_flat, pe128, emb128)
    out_t = _project_tc(xpe, W.T)
    return jnp.transpose(out_t, (0, 2, 1))
